# baseline (speedup 1.0000x reference)
import functools

import jax
import jax.numpy as jnp
import numpy as np
from jax import lax
from jax.experimental import pallas as pl

_NT = 1000
_B = 64
_C = 3
_N = 2048
_BBLK = 8
_GRID = _B // _BBLK


def _make_coef_table() -> np.ndarray:
    betas = np.linspace(0.0001, 0.02, _NT).astype(np.float64)
    alphas = 1.0 - betas
    ac = np.cumprod(alphas, axis=0)
    acp = np.append(1.0, ac[:-1])
    sra = np.sqrt(1.0 / ac)
    srm1 = np.sqrt(1.0 / ac - 1.0)
    pv = betas * (1.0 - acp) / (1.0 - ac)
    c1 = betas * np.sqrt(acp) / (1.0 - ac)
    c2 = (1.0 - acp) * np.sqrt(alphas) / (1.0 - ac)
    plvc32 = np.log(np.maximum(pv, 1e-20)).astype(np.float32)
    sig = np.exp(0.5 * plvc32.astype(np.float64))
    sig[0] = 0.0
    tbl = np.zeros((_NT, 8), dtype=np.float32)
    tbl[:, 0] = sra
    tbl[:, 1] = srm1
    tbl[:, 2] = c1
    tbl[:, 3] = c2
    tbl[:, 4] = sig
    return tbl


_COEF = _make_coef_table()


def _p_sample_body(t_ref, coef_ref, d_ref, m_ref, z_ref, s_ref, x_ref):
    tb = t_ref[...]
    oh = (tb == lax.broadcasted_iota(jnp.int32, (_BBLK, _NT), 1)
          ).astype(jnp.float32)
    cf = jnp.dot(oh, coef_ref[...], preferred_element_type=jnp.float32,
                 precision=lax.Precision.HIGHEST)
    d = d_ref[...]
    m = m_ref[...]
    z = z_ref[...]
    sra = cf[:, 0][:, None, None]
    srm1 = cf[:, 1][:, None, None]
    c1 = cf[:, 2][:, None, None]
    c2 = cf[:, 3][:, None, None]
    sg = cf[:, 4][:, None, None]
    xr = sra * d - srm1 * m
    xr = jnp.minimum(jnp.maximum(xr, -0.5), 0.5)
    x_ref[...] = xr
    s_ref[...] = c1 * xr + c2 * d + sg * z


@jax.jit
def _p_sample(data, t2, model_output, noise, coef):
    blk = pl.BlockSpec((_BBLK, _C, _N), lambda i: (i, 0, 0))
    return pl.pallas_call(
        _p_sample_body,
        grid=(_GRID,),
        in_specs=[
            pl.BlockSpec((_BBLK, 1), lambda i: (i, 0)),
            pl.BlockSpec((_NT, 8), lambda i: (0, 0)),
            blk,
            blk,
            blk,
        ],
        out_specs=[blk, blk],
        out_shape=[
            jax.ShapeDtypeStruct((_B, _C, _N), jnp.float32),
            jax.ShapeDtypeStruct((_B, _C, _N), jnp.float32),
        ],
    )(t2, coef, data, model_output, noise)


def kernel(data, t, model_output, noise):
    t2 = t.astype(jnp.int32)[:, None]
    sample, xrec = _p_sample(data, t2, model_output, noise,
                             jnp.asarray(_COEF))
    return sample, xrec


# speedup vs baseline: 2.3604x; 2.3604x over previous
import functools

import jax
import jax.numpy as jnp
import numpy as np
from jax import lax
from jax.experimental import pallas as pl

_NT = 1000
_B = 64
_C = 3
_N = 2048
_BBLK = 8
_GRID = _B // _BBLK


def _make_coef_table() -> np.ndarray:
    betas = np.linspace(0.0001, 0.02, _NT).astype(np.float64)
    alphas = 1.0 - betas
    ac = np.cumprod(alphas, axis=0)
    acp = np.append(1.0, ac[:-1])
    sra = np.sqrt(1.0 / ac)
    srm1 = np.sqrt(1.0 / ac - 1.0)
    pv = betas * (1.0 - acp) / (1.0 - ac)
    c1 = betas * np.sqrt(acp) / (1.0 - ac)
    c2 = (1.0 - acp) * np.sqrt(alphas) / (1.0 - ac)
    plvc32 = np.log(np.maximum(pv, 1e-20)).astype(np.float32)
    sig = np.exp(0.5 * plvc32.astype(np.float64))
    sig[0] = 0.0
    tbl = np.zeros((_NT, 8), dtype=np.float32)
    tbl[:, 0] = sra
    tbl[:, 1] = srm1
    tbl[:, 2] = c1
    tbl[:, 3] = c2
    tbl[:, 4] = sig
    return tbl


_COEF = _make_coef_table()


def _p_sample_body(t_ref, coef_ref, d_ref, m_ref, z_ref, s_ref, x_ref):
    tb = t_ref[...]
    oh = (tb == lax.broadcasted_iota(jnp.int32, (_BBLK, _NT), 1)
          ).astype(jnp.float32)
    cf = jnp.dot(oh, coef_ref[...], preferred_element_type=jnp.float32,
                 precision=lax.Precision.HIGHEST)
    d = d_ref[...]
    m = m_ref[...]
    z = z_ref[...]
    sra = cf[:, 0:1][None]
    srm1 = cf[:, 1:2][None]
    c1 = cf[:, 2:3][None]
    c2 = cf[:, 3:4][None]
    sg = cf[:, 4:5][None]
    xr = sra * d - srm1 * m
    xr = jnp.minimum(jnp.maximum(xr, -0.5), 0.5)
    x_ref[...] = xr
    s_ref[...] = c1 * xr + c2 * d + sg * z


@jax.jit
def _p_sample(data, t2, model_output, noise, coef):
    blk = pl.BlockSpec((_C, _BBLK, _N), lambda i: (0, i, 0))
    return pl.pallas_call(
        _p_sample_body,
        grid=(_GRID,),
        in_specs=[
            pl.BlockSpec((_BBLK, 1), lambda i: (i, 0)),
            pl.BlockSpec((_NT, 8), lambda i: (0, 0)),
            blk,
            blk,
            blk,
        ],
        out_specs=[blk, blk],
        out_shape=[
            jax.ShapeDtypeStruct((_C, _B, _N), jnp.float32),
            jax.ShapeDtypeStruct((_C, _B, _N), jnp.float32),
        ],
    )(t2, coef, data, model_output, noise)


def kernel(data, t, model_output, noise):
    dt = jnp.transpose(data, (1, 0, 2))
    mt = jnp.transpose(model_output, (1, 0, 2))
    nt = jnp.transpose(noise, (1, 0, 2))
    t2 = t.astype(jnp.int32)[:, None]
    sample, xrec = _p_sample(dt, t2, mt, nt, jnp.asarray(_COEF))
    return jnp.transpose(sample, (1, 0, 2)), jnp.transpose(xrec, (1, 0, 2))


# speedup vs baseline: 5.0479x; 2.1385x over previous
import jax
import jax.numpy as jnp
import numpy as np
from jax import lax
from jax.experimental import pallas as pl
from jax.experimental.pallas import tpu as pltpu

_NT = 1000
_B = 64
_C = 3
_N = 2048
_BBLK = 32
_GRID = _B // _BBLK


def _make_coef_table() -> np.ndarray:
    betas = np.linspace(0.0001, 0.02, _NT).astype(np.float64)
    alphas = 1.0 - betas
    ac = np.cumprod(alphas, axis=0)
    acp = np.append(1.0, ac[:-1])
    sra = np.sqrt(1.0 / ac)
    srm1 = np.sqrt(1.0 / ac - 1.0)
    pv = betas * (1.0 - acp) / (1.0 - ac)
    c1 = betas * np.sqrt(acp) / (1.0 - ac)
    c2 = (1.0 - acp) * np.sqrt(alphas) / (1.0 - ac)
    plvc32 = np.log(np.maximum(pv, 1e-20)).astype(np.float32)
    sig = np.exp(0.5 * plvc32.astype(np.float64))
    sig[0] = 0.0
    tbl = np.zeros((_NT, 8), dtype=np.float32)
    tbl[:, 0] = sra
    tbl[:, 1] = srm1
    tbl[:, 2] = c1
    tbl[:, 3] = c2
    tbl[:, 4] = sig
    return tbl


_COEF = _make_coef_table()


def _p_sample_body(t_ref, coef_ref, d_ref, m_ref, z_ref, s_ref, x_ref,
                   cf_ref):
    i = pl.program_id(0)

    @pl.when(i == 0)
    def _():
        tb = t_ref[...]
        oh = (tb == lax.broadcasted_iota(jnp.int32, (_NT, _B), 0)
              ).astype(jnp.float32)
        cf_ref[...] = lax.dot_general(
            oh, coef_ref[...], (((0,), (0,)), ((), ())),
            preferred_element_type=jnp.float32,
            precision=lax.Precision.HIGHEST)

    cf = cf_ref[pl.ds(i * _BBLK, _BBLK), :]
    d = d_ref[...]
    m = m_ref[...]
    z = z_ref[...]
    sra = cf[:, 0:1][None]
    srm1 = cf[:, 1:2][None]
    c1 = cf[:, 2:3][None]
    c2 = cf[:, 3:4][None]
    sg = cf[:, 4:5][None]
    xr = sra * d - srm1 * m
    xr = jnp.minimum(jnp.maximum(xr, -0.5), 0.5)
    x_ref[...] = xr
    s_ref[...] = c1 * xr + c2 * d + sg * z


@jax.jit
def _p_sample(data, t2, model_output, noise, coef):
    blk = pl.BlockSpec((_C, _BBLK, _N), lambda i: (0, i, 0))
    return pl.pallas_call(
        _p_sample_body,
        grid=(_GRID,),
        in_specs=[
            pl.BlockSpec((1, _B), lambda i: (0, 0)),
            pl.BlockSpec((_NT, 8), lambda i: (0, 0)),
            blk,
            blk,
            blk,
        ],
        scratch_shapes=[pltpu.VMEM((_B, 8), jnp.float32)],
        out_specs=[blk, blk],
        out_shape=[
            jax.ShapeDtypeStruct((_C, _B, _N), jnp.float32),
            jax.ShapeDtypeStruct((_C, _B, _N), jnp.float32),
        ],
    )(t2, coef, data, model_output, noise)


def kernel(data, t, model_output, noise):
    dt = jnp.transpose(data, (1, 0, 2))
    mt = jnp.transpose(model_output, (1, 0, 2))
    nt = jnp.transpose(noise, (1, 0, 2))
    t2 = t.astype(jnp.int32)[None, :]
    sample, xrec = _p_sample(dt, t2, mt, nt, jnp.asarray(_COEF))
    return jnp.transpose(sample, (1, 0, 2)), jnp.transpose(xrec, (1, 0, 2))
